# baseline (speedup 1.0000x reference)
import jax
import jax.numpy as jnp
from jax import lax
from jax.experimental import pallas as pl
from jax.experimental.pallas import tpu as pltpu

D_MODEL = 1024
D_STATE = 16
D_CONV = 4
D_INNER = 2048
EPS = 1e-5

B = 2
L = 1024
M_TILE = 256
T_CHUNK = 256
N_CHUNKS = L // T_CHUNK
GROUP = 8


def _ln_proj_kernel(x_ref, g_ref, b_ref, w_ref, o_ref):
    x = x_ref[...]
    mu = jnp.mean(x, axis=-1, keepdims=True)
    xc = x - mu
    var = jnp.mean(xc * xc, axis=-1, keepdims=True)
    xn = (xc * lax.rsqrt(var + EPS) * g_ref[...] + b_ref[...]
          ).astype(jnp.bfloat16)
    o_ref[...] = jnp.dot(xn, w_ref[...],
                         preferred_element_type=jnp.float32
                         ).astype(jnp.bfloat16)


def _ln_proj(x2d, ln_g, ln_b, W_in):
    n_m = (B * L) // M_TILE
    return pl.pallas_call(
        _ln_proj_kernel,
        grid=(2, n_m),
        in_specs=[
            pl.BlockSpec((M_TILE, D_MODEL), lambda j, i: (i, 0)),
            pl.BlockSpec((1, D_MODEL), lambda j, i: (0, 0)),
            pl.BlockSpec((1, D_MODEL), lambda j, i: (0, 0)),
            pl.BlockSpec((D_MODEL, D_INNER), lambda j, i: (0, j)),
        ],
        out_specs=pl.BlockSpec((M_TILE, D_INNER), lambda j, i: (i, j)),
        out_shape=jax.ShapeDtypeStruct((B * L, 2 * D_INNER), jnp.bfloat16),
        compiler_params=pltpu.CompilerParams(
            dimension_semantics=("parallel", "arbitrary"),
            vmem_limit_bytes=56 * 1024 * 1024,
        ),
    )(x2d, ln_g, ln_b, W_in)


N_GROUPS = T_CHUNK // GROUP
LAST_SLOT = (N_GROUPS - 1) & 1


def _fused_kernel(xr_ref, z_ref, x_ref, cw_ref, cb_ref, wd_ref, bd_ref,
                  wb_ref, wc_ref, at_ref, dp_ref, wo_ref, o_ref,
                  h_ref, tail_ref, xb_ref, dl_ref, dx_ref, ys_ref, ps_ref):
    l = pl.program_id(1)

    @pl.when(l == 0)
    def _():
        h_ref[...] = jnp.zeros_like(h_ref)
        tail_ref[...] = jnp.zeros_like(tail_ref)

    xr = xr_ref[...].astype(jnp.float32)
    acc = xr * cw_ref[D_CONV - 1:D_CONV, :] + cb_ref[...]
    for k in range(1, D_CONV):
        sh = jnp.concatenate(
            [tail_ref[GROUP - k:GROUP, :], xr[: T_CHUNK - k, :]], axis=0)
        acc = acc + sh * cw_ref[D_CONV - 1 - k:D_CONV - k, :]
    tail_ref[...] = xr[T_CHUNK - GROUP:, :]
    xb = acc * (1.0 / (1.0 + jnp.exp(-acc)))
    xb_ref[...] = xb

    dmat = jnp.dot(xb, wd_ref[...],
                   preferred_element_type=jnp.float32) + bd_ref[...]
    delta = jnp.maximum(dmat, 0.0) + jnp.log1p(jnp.exp(-jnp.abs(dmat)))
    dl_ref[...] = delta
    dx_ref[...] = delta * xb
    bt = lax.dot_general(wb_ref[...], xb, (((0,), (1,)), ((), ())),
                         preferred_element_type=jnp.float32)
    ct = lax.dot_general(wc_ref[...], xb, (((0,), (1,)), ((), ())),
                         preferred_element_type=jnp.float32)
    at = at_ref[...]
    dp = dp_ref[...]

    sel = (lax.broadcasted_iota(jnp.int32, (GROUP, GROUP * D_STATE), 1)
           // D_STATE
           == lax.broadcasted_iota(jnp.int32, (GROUP, GROUP * D_STATE), 0)
           ).astype(jnp.float32)

    def group(g, h):
        g8 = pl.multiple_of(g * GROUP, GROUP)
        gp8 = pl.multiple_of((jnp.maximum(g, 1) - 1) * GROUP, GROUP)
        slot = g & 1
        y8 = jnp.dot(sel, ps_ref[1 - slot],
                     preferred_element_type=jnp.float32)
        dl8 = dl_ref[pl.ds(g8, GROUP), :]
        dx8 = dx_ref[pl.ds(g8, GROUP), :]
        shift = (T_CHUNK - g8) & (T_CHUNK - 1)
        bt8 = pltpu.roll(bt, shift, axis=1)[:, :GROUP]
        ct8 = pltpu.roll(ct, shift, axis=1)[:, :GROUP]
        for r in range(GROUP):
            a = jnp.exp2(at * dl8[r:r + 1, :])
            u = bt8[:, r:r + 1] * dx8[r:r + 1, :]
            h = a * h + u
            ps_ref[slot, r * D_STATE:(r + 1) * D_STATE, :] = \
                h * ct8[:, r:r + 1]

        @pl.when(g > 0)
        def _():
            ys_ref[pl.ds(gp8, GROUP), :] = y8
        return h

    h = lax.fori_loop(0, N_GROUPS, group, h_ref[...])
    h_ref[...] = h
    yl = jnp.dot(sel, ps_ref[LAST_SLOT], preferred_element_type=jnp.float32)
    ys_ref[pl.ds(T_CHUNK - GROUP, GROUP), :] = yl

    z = z_ref[...].astype(jnp.float32)
    g = ((ys_ref[...] + xb_ref[...] * dp) * (z * (1.0 / (1.0 + jnp.exp(-z))))
         ).astype(jnp.bfloat16)
    o_ref[...] = jnp.dot(g, wo_ref[...],
                         preferred_element_type=jnp.float32) + x_ref[...]


def _fused(xz2d, x2d, cw, cb, W_delta, b_delta, W_b, W_c, A_t, Dp2, W_out):
    return pl.pallas_call(
        _fused_kernel,
        grid=(B, N_CHUNKS),
        in_specs=[
            pl.BlockSpec((T_CHUNK, D_INNER), lambda b, l: (b * N_CHUNKS + l, 0)),
            pl.BlockSpec((T_CHUNK, D_INNER), lambda b, l: (b * N_CHUNKS + l, 1)),
            pl.BlockSpec((T_CHUNK, D_MODEL), lambda b, l: (b * N_CHUNKS + l, 0)),
            pl.BlockSpec((D_CONV, D_INNER), lambda b, l: (0, 0)),
            pl.BlockSpec((1, D_INNER), lambda b, l: (0, 0)),
            pl.BlockSpec((D_INNER, D_INNER), lambda b, l: (0, 0)),
            pl.BlockSpec((1, D_INNER), lambda b, l: (0, 0)),
            pl.BlockSpec((D_INNER, D_STATE), lambda b, l: (0, 0)),
            pl.BlockSpec((D_INNER, D_STATE), lambda b, l: (0, 0)),
            pl.BlockSpec((D_STATE, D_INNER), lambda b, l: (0, 0)),
            pl.BlockSpec((1, D_INNER), lambda b, l: (0, 0)),
            pl.BlockSpec((D_INNER, D_MODEL), lambda b, l: (0, 0)),
        ],
        out_specs=pl.BlockSpec((T_CHUNK, D_MODEL), lambda b, l: (b * N_CHUNKS + l, 0)),
        out_shape=jax.ShapeDtypeStruct((B * L, D_MODEL), jnp.float32),
        scratch_shapes=[
            pltpu.VMEM((D_STATE, D_INNER), jnp.float32),
            pltpu.VMEM((GROUP, D_INNER), jnp.float32),
            pltpu.VMEM((T_CHUNK, D_INNER), jnp.float32),
            pltpu.VMEM((T_CHUNK, D_INNER), jnp.float32),
            pltpu.VMEM((T_CHUNK, D_INNER), jnp.float32),
            pltpu.VMEM((T_CHUNK, D_INNER), jnp.float32),
            pltpu.VMEM((2, GROUP * D_STATE, D_INNER), jnp.float32),
        ],
        compiler_params=pltpu.CompilerParams(
            dimension_semantics=("arbitrary", "arbitrary"),
            vmem_limit_bytes=56 * 1024 * 1024,
        ),
    )(xz2d, xz2d, x2d, cw, cb, W_delta, b_delta, W_b, W_c, A_t, Dp2, W_out)


def kernel(x, ln_g, ln_b, W_in, conv_w, conv_b, W_b, W_c, W_delta, b_delta,
           log_A, Dp, W_out):
    x2d = x.reshape(B * L, D_MODEL)
    xz = _ln_proj(x2d, ln_g.reshape(1, -1), ln_b.reshape(1, -1),
                  W_in.astype(jnp.bfloat16))
    cw = jnp.transpose(conv_w[:, 0, :])
    A_t = jnp.transpose(-jnp.exp(log_A)) * 1.4426950408889634
    out2d = _fused(xz, x2d, cw, conv_b.reshape(1, -1), W_delta,
                   b_delta.reshape(1, -1), W_b, W_c, A_t,
                   Dp.reshape(1, -1), W_out.astype(jnp.bfloat16))
    return out2d.reshape(B, L, D_MODEL)


# speedup vs baseline: 1.1135x; 1.1135x over previous
import jax
import jax.numpy as jnp
from jax import lax
from jax.experimental import pallas as pl
from jax.experimental.pallas import tpu as pltpu

D_MODEL = 1024
D_STATE = 16
D_CONV = 4
D_INNER = 2048
EPS = 1e-5

B = 2
L = 1024
M_TILE = 256
T_CHUNK = 256
N_CHUNKS = L // T_CHUNK
GROUP = 16


def _ln_proj_kernel(x_ref, g_ref, b_ref, w_ref, o_ref):
    x = x_ref[...]
    mu = jnp.mean(x, axis=-1, keepdims=True)
    xc = x - mu
    var = jnp.mean(xc * xc, axis=-1, keepdims=True)
    xn = xc * lax.rsqrt(var + EPS) * g_ref[...] + b_ref[...]
    o_ref[...] = jnp.dot(xn, w_ref[...], preferred_element_type=jnp.float32)


def _ln_proj(x2d, ln_g, ln_b, W_in):
    n_m = (B * L) // M_TILE
    return pl.pallas_call(
        _ln_proj_kernel,
        grid=(2, n_m),
        in_specs=[
            pl.BlockSpec((M_TILE, D_MODEL), lambda j, i: (i, 0)),
            pl.BlockSpec((1, D_MODEL), lambda j, i: (0, 0)),
            pl.BlockSpec((1, D_MODEL), lambda j, i: (0, 0)),
            pl.BlockSpec((D_MODEL, D_INNER), lambda j, i: (0, j)),
        ],
        out_specs=pl.BlockSpec((M_TILE, D_INNER), lambda j, i: (i, j)),
        out_shape=jax.ShapeDtypeStruct((B * L, 2 * D_INNER), jnp.float32),
        compiler_params=pltpu.CompilerParams(
            dimension_semantics=("parallel", "arbitrary"),
            vmem_limit_bytes=56 * 1024 * 1024,
        ),
    )(x2d, ln_g, ln_b, W_in)


N_GROUPS = T_CHUNK // GROUP
LAST_SLOT = (N_GROUPS - 1) & 1


def _fused_kernel(xr_ref, z_ref, x_ref, cw_ref, cb_ref, wd_ref, bd_ref,
                  wb_ref, wc_ref, at_ref, dp_ref, wo_ref, o_ref,
                  h_ref, tail_ref, xb_ref, dl_ref, dx_ref, ys_ref, ps_ref):
    l = pl.program_id(1)

    @pl.when(l == 0)
    def _():
        h_ref[...] = jnp.zeros_like(h_ref)
        tail_ref[...] = jnp.zeros_like(tail_ref)

    xr = xr_ref[...]
    acc = xr * cw_ref[D_CONV - 1:D_CONV, :] + cb_ref[...]
    for k in range(1, D_CONV):
        sh = jnp.concatenate(
            [tail_ref[GROUP - k:GROUP, :], xr[: T_CHUNK - k, :]], axis=0)
        acc = acc + sh * cw_ref[D_CONV - 1 - k:D_CONV - k, :]
    tail_ref[...] = xr[T_CHUNK - GROUP:, :]
    xb = acc * (1.0 / (1.0 + jnp.exp(-acc)))
    xb_ref[...] = xb

    dmat = jnp.dot(xb, wd_ref[...],
                   preferred_element_type=jnp.float32) + bd_ref[...]
    delta = jnp.maximum(dmat, 0.0) + jnp.log1p(jnp.exp(-jnp.abs(dmat)))
    dl_ref[...] = delta
    dx_ref[...] = delta * xb
    bt = lax.dot_general(wb_ref[...], xb, (((0,), (1,)), ((), ())),
                         preferred_element_type=jnp.float32)
    ct = lax.dot_general(wc_ref[...], xb, (((0,), (1,)), ((), ())),
                         preferred_element_type=jnp.float32)
    at = at_ref[...]
    dp = dp_ref[...]

    sel = (lax.broadcasted_iota(jnp.int32, (GROUP, GROUP * D_STATE), 1)
           // D_STATE
           == lax.broadcasted_iota(jnp.int32, (GROUP, GROUP * D_STATE), 0)
           ).astype(jnp.float32)

    def group(g, h):
        g8 = pl.multiple_of(g * GROUP, GROUP)
        gp8 = pl.multiple_of((jnp.maximum(g, 1) - 1) * GROUP, GROUP)
        slot = g & 1
        y8 = jnp.dot(sel, ps_ref[1 - slot],
                     preferred_element_type=jnp.float32)
        ys_ref[pl.ds(gp8, GROUP), :] = y8
        dl8 = dl_ref[pl.ds(g8, GROUP), :]
        dx8 = dx_ref[pl.ds(g8, GROUP), :]
        shift = (T_CHUNK - g8) & (T_CHUNK - 1)
        bt8 = pltpu.roll(bt, shift, axis=1)[:, :GROUP]
        ct8 = pltpu.roll(ct, shift, axis=1)[:, :GROUP]
        for r in range(GROUP):
            a = jnp.exp2(at * dl8[r:r + 1, :])
            u = bt8[:, r:r + 1] * dx8[r:r + 1, :]
            h = a * h + u
            ps_ref[slot, r * D_STATE:(r + 1) * D_STATE, :] = \
                h * ct8[:, r:r + 1]
        return h

    h = lax.fori_loop(0, N_GROUPS, group, h_ref[...])
    h_ref[...] = h
    yl = jnp.dot(sel, ps_ref[LAST_SLOT], preferred_element_type=jnp.float32)
    ys_ref[pl.ds(T_CHUNK - GROUP, GROUP), :] = yl

    z = z_ref[...]
    g = (ys_ref[...] + xb_ref[...] * dp) * (z * (1.0 / (1.0 + jnp.exp(-z))))
    o_ref[...] = jnp.dot(g, wo_ref[...],
                         preferred_element_type=jnp.float32) + x_ref[...]


def _fused(xz2d, x2d, cw, cb, W_delta, b_delta, W_b, W_c, A_t, Dp2, W_out):
    return pl.pallas_call(
        _fused_kernel,
        grid=(B, N_CHUNKS),
        in_specs=[
            pl.BlockSpec((T_CHUNK, D_INNER), lambda b, l: (b * N_CHUNKS + l, 0)),
            pl.BlockSpec((T_CHUNK, D_INNER), lambda b, l: (b * N_CHUNKS + l, 1)),
            pl.BlockSpec((T_CHUNK, D_MODEL), lambda b, l: (b * N_CHUNKS + l, 0)),
            pl.BlockSpec((D_CONV, D_INNER), lambda b, l: (0, 0)),
            pl.BlockSpec((1, D_INNER), lambda b, l: (0, 0)),
            pl.BlockSpec((D_INNER, D_INNER), lambda b, l: (0, 0)),
            pl.BlockSpec((1, D_INNER), lambda b, l: (0, 0)),
            pl.BlockSpec((D_INNER, D_STATE), lambda b, l: (0, 0)),
            pl.BlockSpec((D_INNER, D_STATE), lambda b, l: (0, 0)),
            pl.BlockSpec((D_STATE, D_INNER), lambda b, l: (0, 0)),
            pl.BlockSpec((1, D_INNER), lambda b, l: (0, 0)),
            pl.BlockSpec((D_INNER, D_MODEL), lambda b, l: (0, 0)),
        ],
        out_specs=pl.BlockSpec((T_CHUNK, D_MODEL), lambda b, l: (b * N_CHUNKS + l, 0)),
        out_shape=jax.ShapeDtypeStruct((B * L, D_MODEL), jnp.float32),
        scratch_shapes=[
            pltpu.VMEM((D_STATE, D_INNER), jnp.float32),
            pltpu.VMEM((GROUP, D_INNER), jnp.float32),
            pltpu.VMEM((T_CHUNK, D_INNER), jnp.float32),
            pltpu.VMEM((T_CHUNK, D_INNER), jnp.float32),
            pltpu.VMEM((T_CHUNK, D_INNER), jnp.float32),
            pltpu.VMEM((T_CHUNK, D_INNER), jnp.float32),
            pltpu.VMEM((2, GROUP * D_STATE, D_INNER), jnp.float32),
        ],
        compiler_params=pltpu.CompilerParams(
            dimension_semantics=("arbitrary", "arbitrary"),
            vmem_limit_bytes=56 * 1024 * 1024,
        ),
    )(xz2d, xz2d, x2d, cw, cb, W_delta, b_delta, W_b, W_c, A_t, Dp2, W_out)


def kernel(x, ln_g, ln_b, W_in, conv_w, conv_b, W_b, W_c, W_delta, b_delta,
           log_A, Dp, W_out):
    x2d = x.reshape(B * L, D_MODEL)
    xz = _ln_proj(x2d, ln_g.reshape(1, -1), ln_b.reshape(1, -1), W_in)
    cw = jnp.transpose(conv_w[:, 0, :])
    A_t = jnp.transpose(-jnp.exp(log_A)) * 1.4426950408889634
    out2d = _fused(xz, x2d, cw, conv_b.reshape(1, -1), W_delta,
                   b_delta.reshape(1, -1), W_b, W_c, A_t,
                   Dp.reshape(1, -1), W_out)
    return out2d.reshape(B, L, D_MODEL)


# speedup vs baseline: 1.1234x; 1.0089x over previous
import jax
import jax.numpy as jnp
from jax import lax
from jax.experimental import pallas as pl
from jax.experimental.pallas import tpu as pltpu

D_MODEL = 1024
D_STATE = 16
D_CONV = 4
D_INNER = 2048
EPS = 1e-5

B = 2
L = 1024
M_TILE = 256
T_CHUNK = 256
N_CHUNKS = L // T_CHUNK
GROUP = 16


def _ln_proj_kernel(x_ref, g_ref, b_ref, w_ref, o_ref):
    x = x_ref[...]
    mu = jnp.mean(x, axis=-1, keepdims=True)
    xc = x - mu
    var = jnp.mean(xc * xc, axis=-1, keepdims=True)
    xn = xc * lax.rsqrt(var + EPS) * g_ref[...] + b_ref[...]
    o_ref[...] = jnp.dot(xn, w_ref[...], preferred_element_type=jnp.float32)


def _ln_proj(x2d, ln_g, ln_b, W_in):
    n_m = (B * L) // M_TILE
    return pl.pallas_call(
        _ln_proj_kernel,
        grid=(2, n_m),
        in_specs=[
            pl.BlockSpec((M_TILE, D_MODEL), lambda j, i: (i, 0)),
            pl.BlockSpec((1, D_MODEL), lambda j, i: (0, 0)),
            pl.BlockSpec((1, D_MODEL), lambda j, i: (0, 0)),
            pl.BlockSpec((D_MODEL, D_INNER), lambda j, i: (0, j)),
        ],
        out_specs=pl.BlockSpec((M_TILE, D_INNER), lambda j, i: (i, j)),
        out_shape=jax.ShapeDtypeStruct((B * L, 2 * D_INNER), jnp.float32),
        compiler_params=pltpu.CompilerParams(
            dimension_semantics=("parallel", "arbitrary"),
            vmem_limit_bytes=56 * 1024 * 1024,
        ),
    )(x2d, ln_g, ln_b, W_in)


N_GROUPS = T_CHUNK // GROUP
LAST_SLOT = (N_GROUPS - 1) & 1


def _fused_kernel(xr_ref, z_ref, x_ref, cw_ref, cb_ref, wd_ref, bd_ref,
                  wb_ref, wc_ref, at_ref, dp_ref, wo_ref, o_ref,
                  h_ref, tail_ref, xb_ref, dl_ref, dx_ref, ct_ref, ys_ref,
                  ps_ref):
    l = pl.program_id(1)

    @pl.when(l == 0)
    def _():
        h_ref[...] = jnp.zeros_like(h_ref)
        tail_ref[...] = jnp.zeros_like(tail_ref)

    xr = xr_ref[...]
    acc = xr * cw_ref[D_CONV - 1:D_CONV, :] + cb_ref[...]
    for k in range(1, D_CONV):
        sh = jnp.concatenate(
            [tail_ref[GROUP - k:GROUP, :], xr[: T_CHUNK - k, :]], axis=0)
        acc = acc + sh * cw_ref[D_CONV - 1 - k:D_CONV - k, :]
    tail_ref[...] = xr[T_CHUNK - GROUP:, :]
    xb = acc * (1.0 / (1.0 + jnp.exp(-acc)))
    xb_ref[...] = xb

    dmat = jnp.dot(xb, wd_ref[...],
                   preferred_element_type=jnp.float32) + bd_ref[...]
    delta = jnp.maximum(dmat, 0.0) + jnp.log1p(jnp.exp(-jnp.abs(dmat)))
    dl_ref[...] = delta
    dx_ref[...] = delta * xb
    bt = lax.dot_general(wb_ref[...], xb, (((0,), (1,)), ((), ())),
                         preferred_element_type=jnp.float32)
    ct_ref[...] = jnp.dot(xb, wc_ref[...],
                          preferred_element_type=jnp.float32)
    at = at_ref[...]
    dp = dp_ref[...]

    diag = (lax.broadcasted_iota(jnp.int32, (GROUP, GROUP * D_STATE), 1)
            // D_STATE
            == lax.broadcasted_iota(jnp.int32, (GROUP, GROUP * D_STATE), 0)
            ).astype(jnp.float32)

    def group(g, h):
        g8 = pl.multiple_of(g * GROUP, GROUP)
        gp8 = pl.multiple_of((jnp.maximum(g, 1) - 1) * GROUP, GROUP)
        slot = g & 1
        gp8v = pl.multiple_of((jnp.maximum(g, 1) - 1) * GROUP, GROUP)
        ct8t = ct_ref[pl.ds(gp8v, GROUP), :]
        s_c = diag * pltpu.repeat(ct8t, GROUP, axis=1)
        y8 = jnp.dot(s_c, ps_ref[1 - slot],
                     preferred_element_type=jnp.float32)
        ys_ref[pl.ds(gp8, GROUP), :] = y8
        dl8 = dl_ref[pl.ds(g8, GROUP), :]
        dx8 = dx_ref[pl.ds(g8, GROUP), :]
        shift = (T_CHUNK - g8) & (T_CHUNK - 1)
        bt8 = pltpu.roll(bt, shift, axis=1)[:, :GROUP]
        for r in range(GROUP):
            a = jnp.exp2(at * dl8[r:r + 1, :])
            u = bt8[:, r:r + 1] * dx8[r:r + 1, :]
            h = a * h + u
            ps_ref[slot, r * D_STATE:(r + 1) * D_STATE, :] = h
        return h

    h = lax.fori_loop(0, N_GROUPS, group, h_ref[...])
    h_ref[...] = h
    ctlt = ct_ref[pl.ds(T_CHUNK - GROUP, GROUP), :]
    s_cl = diag * pltpu.repeat(ctlt, GROUP, axis=1)
    yl = jnp.dot(s_cl, ps_ref[LAST_SLOT], preferred_element_type=jnp.float32)
    ys_ref[pl.ds(T_CHUNK - GROUP, GROUP), :] = yl

    z = z_ref[...]
    g = (ys_ref[...] + xb_ref[...] * dp) * (z * (1.0 / (1.0 + jnp.exp(-z))))
    o_ref[...] = jnp.dot(g, wo_ref[...],
                         preferred_element_type=jnp.float32) + x_ref[...]


def _fused(xz2d, x2d, cw, cb, W_delta, b_delta, W_b, W_c, A_t, Dp2, W_out):
    return pl.pallas_call(
        _fused_kernel,
        grid=(B, N_CHUNKS),
        in_specs=[
            pl.BlockSpec((T_CHUNK, D_INNER), lambda b, l: (b * N_CHUNKS + l, 0)),
            pl.BlockSpec((T_CHUNK, D_INNER), lambda b, l: (b * N_CHUNKS + l, 1)),
            pl.BlockSpec((T_CHUNK, D_MODEL), lambda b, l: (b * N_CHUNKS + l, 0)),
            pl.BlockSpec((D_CONV, D_INNER), lambda b, l: (0, 0)),
            pl.BlockSpec((1, D_INNER), lambda b, l: (0, 0)),
            pl.BlockSpec((D_INNER, D_INNER), lambda b, l: (0, 0)),
            pl.BlockSpec((1, D_INNER), lambda b, l: (0, 0)),
            pl.BlockSpec((D_INNER, D_STATE), lambda b, l: (0, 0)),
            pl.BlockSpec((D_INNER, D_STATE), lambda b, l: (0, 0)),
            pl.BlockSpec((D_STATE, D_INNER), lambda b, l: (0, 0)),
            pl.BlockSpec((1, D_INNER), lambda b, l: (0, 0)),
            pl.BlockSpec((D_INNER, D_MODEL), lambda b, l: (0, 0)),
        ],
        out_specs=pl.BlockSpec((T_CHUNK, D_MODEL), lambda b, l: (b * N_CHUNKS + l, 0)),
        out_shape=jax.ShapeDtypeStruct((B * L, D_MODEL), jnp.float32),
        scratch_shapes=[
            pltpu.VMEM((D_STATE, D_INNER), jnp.float32),
            pltpu.VMEM((GROUP, D_INNER), jnp.float32),
            pltpu.VMEM((T_CHUNK, D_INNER), jnp.float32),
            pltpu.VMEM((T_CHUNK, D_INNER), jnp.float32),
            pltpu.VMEM((T_CHUNK, D_INNER), jnp.float32),
            pltpu.VMEM((T_CHUNK, D_STATE), jnp.float32),
            pltpu.VMEM((T_CHUNK, D_INNER), jnp.float32),
            pltpu.VMEM((2, GROUP * D_STATE, D_INNER), jnp.float32),
        ],
        compiler_params=pltpu.CompilerParams(
            dimension_semantics=("arbitrary", "arbitrary"),
            vmem_limit_bytes=56 * 1024 * 1024,
        ),
    )(xz2d, xz2d, x2d, cw, cb, W_delta, b_delta, W_b, W_c, A_t, Dp2, W_out)


def kernel(x, ln_g, ln_b, W_in, conv_w, conv_b, W_b, W_c, W_delta, b_delta,
           log_A, Dp, W_out):
    x2d = x.reshape(B * L, D_MODEL)
    xz = _ln_proj(x2d, ln_g.reshape(1, -1), ln_b.reshape(1, -1), W_in)
    cw = jnp.transpose(conv_w[:, 0, :])
    A_t = jnp.transpose(-jnp.exp(log_A)) * 1.4426950408889634
    out2d = _fused(xz, x2d, cw, conv_b.reshape(1, -1), W_delta,
                   b_delta.reshape(1, -1), W_b, W_c, A_t,
                   Dp.reshape(1, -1), W_out)
    return out2d.reshape(B, L, D_MODEL)


# speedup vs baseline: 1.1449x; 1.0191x over previous
import jax
import jax.numpy as jnp
from jax import lax
from jax.experimental import pallas as pl
from jax.experimental.pallas import tpu as pltpu

D_MODEL = 1024
D_STATE = 16
D_CONV = 4
D_INNER = 2048
EPS = 1e-5

B = 2
L = 1024
M_TILE = 512
T_CHUNK = 256
N_CHUNKS = L // T_CHUNK
GROUP = 16


def _ln_proj_kernel(x_ref, g_ref, b_ref, w_ref, o_ref):
    x = x_ref[...]
    mu = jnp.mean(x, axis=-1, keepdims=True)
    xc = x - mu
    var = jnp.mean(xc * xc, axis=-1, keepdims=True)
    xn = xc * lax.rsqrt(var + EPS) * g_ref[...] + b_ref[...]
    o_ref[...] = jnp.dot(xn, w_ref[...], preferred_element_type=jnp.float32)


def _ln_proj(x2d, ln_g, ln_b, W_in):
    n_m = (B * L) // M_TILE
    return pl.pallas_call(
        _ln_proj_kernel,
        grid=(2, n_m),
        in_specs=[
            pl.BlockSpec((M_TILE, D_MODEL), lambda j, i: (i, 0)),
            pl.BlockSpec((1, D_MODEL), lambda j, i: (0, 0)),
            pl.BlockSpec((1, D_MODEL), lambda j, i: (0, 0)),
            pl.BlockSpec((D_MODEL, D_INNER), lambda j, i: (0, j)),
        ],
        out_specs=pl.BlockSpec((M_TILE, D_INNER), lambda j, i: (i, j)),
        out_shape=jax.ShapeDtypeStruct((B * L, 2 * D_INNER), jnp.float32),
        compiler_params=pltpu.CompilerParams(
            dimension_semantics=("parallel", "arbitrary"),
            vmem_limit_bytes=56 * 1024 * 1024,
        ),
    )(x2d, ln_g, ln_b, W_in)


N_GROUPS = T_CHUNK // GROUP
LAST_SLOT = (N_GROUPS - 1) & 1


def _fused_kernel(xr_ref, z_ref, x_ref, cw_ref, cb_ref, wd_ref, bd_ref,
                  wb_ref, wc_ref, at_ref, dp_ref, wo_ref, o_ref,
                  h_ref, tail_ref, xb_ref, dl_ref, dx_ref, ct_ref, ys_ref,
                  ps_ref):
    l = pl.program_id(1)

    @pl.when(l == 0)
    def _():
        h_ref[...] = jnp.zeros_like(h_ref)
        tail_ref[...] = jnp.zeros_like(tail_ref)

    xr = xr_ref[...]
    acc = xr * cw_ref[D_CONV - 1:D_CONV, :] + cb_ref[...]
    for k in range(1, D_CONV):
        sh = jnp.concatenate(
            [tail_ref[GROUP - k:GROUP, :], xr[: T_CHUNK - k, :]], axis=0)
        acc = acc + sh * cw_ref[D_CONV - 1 - k:D_CONV - k, :]
    tail_ref[...] = xr[T_CHUNK - GROUP:, :]
    xb = acc * (1.0 / (1.0 + jnp.exp(-acc)))
    xb_ref[...] = xb

    dmat = jnp.dot(xb, wd_ref[...],
                   preferred_element_type=jnp.float32) + bd_ref[...]
    delta = jnp.maximum(dmat, 0.0) + jnp.log1p(jnp.exp(-jnp.abs(dmat)))
    dl_ref[...] = delta
    dx_ref[...] = delta * xb
    bt = lax.dot_general(wb_ref[...], xb, (((0,), (1,)), ((), ())),
                         preferred_element_type=jnp.float32)
    ct_ref[...] = jnp.dot(xb, wc_ref[...],
                          preferred_element_type=jnp.float32)
    at = at_ref[...]
    dp = dp_ref[...]

    diag = (lax.broadcasted_iota(jnp.int32, (GROUP, GROUP * D_STATE), 1)
            // D_STATE
            == lax.broadcasted_iota(jnp.int32, (GROUP, GROUP * D_STATE), 0)
            ).astype(jnp.float32)

    def group(g, h):
        g8 = pl.multiple_of(g * GROUP, GROUP)
        gp8 = pl.multiple_of((jnp.maximum(g, 1) - 1) * GROUP, GROUP)
        slot = g & 1
        gp8v = pl.multiple_of((jnp.maximum(g, 1) - 1) * GROUP, GROUP)
        ct8t = ct_ref[pl.ds(gp8v, GROUP), :]
        s_c = diag * pltpu.repeat(ct8t, GROUP, axis=1)
        y8 = jnp.dot(s_c, ps_ref[1 - slot],
                     preferred_element_type=jnp.float32)
        ys_ref[pl.ds(gp8, GROUP), :] = y8
        dl8 = dl_ref[pl.ds(g8, GROUP), :]
        dx8 = dx_ref[pl.ds(g8, GROUP), :]
        shift = (T_CHUNK - g8) & (T_CHUNK - 1)
        bt8 = pltpu.roll(bt, shift, axis=1)[:, :GROUP]
        for r in range(GROUP):
            a = jnp.exp2(at * dl8[r:r + 1, :])
            u = bt8[:, r:r + 1] * dx8[r:r + 1, :]
            h = a * h + u
            ps_ref[slot, r * D_STATE:(r + 1) * D_STATE, :] = h
        return h

    h = lax.fori_loop(0, N_GROUPS, group, h_ref[...])
    h_ref[...] = h
    ctlt = ct_ref[pl.ds(T_CHUNK - GROUP, GROUP), :]
    s_cl = diag * pltpu.repeat(ctlt, GROUP, axis=1)
    yl = jnp.dot(s_cl, ps_ref[LAST_SLOT], preferred_element_type=jnp.float32)
    ys_ref[pl.ds(T_CHUNK - GROUP, GROUP), :] = yl

    z = z_ref[...]
    g = (ys_ref[...] + xb_ref[...] * dp) * (z * (1.0 / (1.0 + jnp.exp(-z))))
    o_ref[...] = jnp.dot(g, wo_ref[...],
                         preferred_element_type=jnp.float32) + x_ref[...]


def _fused(xz2d, x2d, cw, cb, W_delta, b_delta, W_b, W_c, A_t, Dp2, W_out):
    return pl.pallas_call(
        _fused_kernel,
        grid=(B, N_CHUNKS),
        in_specs=[
            pl.BlockSpec((T_CHUNK, D_INNER), lambda b, l: (b * N_CHUNKS + l, 0)),
            pl.BlockSpec((T_CHUNK, D_INNER), lambda b, l: (b * N_CHUNKS + l, 1)),
            pl.BlockSpec((T_CHUNK, D_MODEL), lambda b, l: (b * N_CHUNKS + l, 0)),
            pl.BlockSpec((D_CONV, D_INNER), lambda b, l: (0, 0)),
            pl.BlockSpec((1, D_INNER), lambda b, l: (0, 0)),
            pl.BlockSpec((D_INNER, D_INNER), lambda b, l: (0, 0)),
            pl.BlockSpec((1, D_INNER), lambda b, l: (0, 0)),
            pl.BlockSpec((D_INNER, D_STATE), lambda b, l: (0, 0)),
            pl.BlockSpec((D_INNER, D_STATE), lambda b, l: (0, 0)),
            pl.BlockSpec((D_STATE, D_INNER), lambda b, l: (0, 0)),
            pl.BlockSpec((1, D_INNER), lambda b, l: (0, 0)),
            pl.BlockSpec((D_INNER, D_MODEL), lambda b, l: (0, 0)),
        ],
        out_specs=pl.BlockSpec((T_CHUNK, D_MODEL), lambda b, l: (b * N_CHUNKS + l, 0)),
        out_shape=jax.ShapeDtypeStruct((B * L, D_MODEL), jnp.float32),
        scratch_shapes=[
            pltpu.VMEM((D_STATE, D_INNER), jnp.float32),
            pltpu.VMEM((GROUP, D_INNER), jnp.float32),
            pltpu.VMEM((T_CHUNK, D_INNER), jnp.float32),
            pltpu.VMEM((T_CHUNK, D_INNER), jnp.float32),
            pltpu.VMEM((T_CHUNK, D_INNER), jnp.float32),
            pltpu.VMEM((T_CHUNK, D_STATE), jnp.float32),
            pltpu.VMEM((T_CHUNK, D_INNER), jnp.float32),
            pltpu.VMEM((2, GROUP * D_STATE, D_INNER), jnp.float32),
        ],
        compiler_params=pltpu.CompilerParams(
            dimension_semantics=("arbitrary", "arbitrary"),
            vmem_limit_bytes=56 * 1024 * 1024,
        ),
    )(xz2d, xz2d, x2d, cw, cb, W_delta, b_delta, W_b, W_c, A_t, Dp2, W_out)


def kernel(x, ln_g, ln_b, W_in, conv_w, conv_b, W_b, W_c, W_delta, b_delta,
           log_A, Dp, W_out):
    x2d = x.reshape(B * L, D_MODEL)
    xz = _ln_proj(x2d, ln_g.reshape(1, -1), ln_b.reshape(1, -1), W_in)
    cw = jnp.transpose(conv_w[:, 0, :])
    A_t = jnp.transpose(-jnp.exp(log_A)) * 1.4426950408889634
    out2d = _fused(xz, x2d, cw, conv_b.reshape(1, -1), W_delta,
                   b_delta.reshape(1, -1), W_b, W_c, A_t,
                   Dp.reshape(1, -1), W_out)
    return out2d.reshape(B, L, D_MODEL)
